# Initial kernel scaffold; baseline (speedup 1.0000x reference)
import jax
import jax.numpy as jnp
from jax.experimental import pallas as pl


def kernel(q, qr, pid, q_table, qr_table, q_diff_table, u_table):
    raise NotImplementedError("write your pallas kernel here")


# speedup vs baseline: 6.5055x; 6.5055x over previous
import functools

import jax
import jax.numpy as jnp
from jax import lax
from jax.experimental import pallas as pl
from jax.experimental.pallas import tpu as pltpu
from jax.experimental.pallas import tpu_sc as plsc

B, L, EMB = 1024, 200, 128
NCHUNK = 2
CHUNK = L // NCHUNK
NW = 32
B_PER_W = B // NW
KV = EMB // 16


def _body(q_hbm, qr_hbm, pid_hbm, qt_hbm, qrt_hbm, qdt_hbm, ut_hbm, out_hbm,
          idx_q, idx_qr, idx_pid, q_rows, qr_rows, qd_rows, u_v, out_v, sem):
    nc = 2
    wid = lax.axis_index("s") * nc + lax.axis_index("c")
    base_b = wid * B_PER_W

    def per_batch(i, _):
        b = base_b + i
        pltpu.sync_copy(q_hbm.at[b], idx_q)
        pltpu.sync_copy(qr_hbm.at[b], idx_qr)
        pltpu.sync_copy(pid_hbm.at[b], idx_pid)
        copies = []
        for j in range(NCHUNK):
            copies.append(pltpu.make_async_copy(
                qt_hbm.at[idx_q.at[j]], q_rows.at[j], sem))
            copies.append(pltpu.make_async_copy(
                qdt_hbm.at[idx_q.at[j]], qd_rows.at[j], sem))
            copies.append(pltpu.make_async_copy(
                qrt_hbm.at[idx_qr.at[j]], qr_rows.at[j], sem))
            copies.append(pltpu.make_async_copy(
                ut_hbm.at[idx_pid.at[j]], u_v.at[pl.ds(j * 128, CHUNK)], sem))
        for c in copies:
            c.start()
        for c in copies:
            c.wait()

        def acc_body(l, accs):
            j, l2 = l // CHUNK, l % CHUNK
            return tuple(a + qr_rows[j, l2, pl.ds(k * 16, 16)]
                         for k, a in enumerate(accs))
        zeros = tuple(jnp.zeros((16,), jnp.float32) for _ in range(KV))
        sums = lax.fori_loop(0, L, acc_body, zeros)
        mean = tuple(s * (1.0 / L) for s in sums)

        for j in range(NCHUNK):
            def out_body(g, _):
                u16 = u_v[pl.ds(j * 128 + g * 4, 16)]
                for i in range(4):
                    l2 = g * 4 + i
                    u = jnp.full((16,), u16[i], jnp.float32)
                    for k in range(KV):
                        ds = pl.ds(k * 16, 16)
                        out_v[j, l2, ds] = (q_rows[j, l2, ds]
                                            + u * qd_rows[j, l2, ds] + mean[k])
                return 0
            lax.fori_loop(0, CHUNK // 4, out_body, 0)

        pltpu.sync_copy(out_v, out_hbm.at[b])
        return 0

    lax.fori_loop(0, B_PER_W, per_batch, 0)


@jax.jit
def _run(q, qr, pid, q_table, qr_table, q_diff_table, u_flat):
    mesh = plsc.VectorSubcoreMesh(core_axis_name="c", subcore_axis_name="s")
    f = pl.kernel(
        _body,
        out_type=jax.ShapeDtypeStruct((B, NCHUNK, CHUNK, EMB), jnp.float32),
        mesh=mesh,
        scratch_types=[
            pltpu.VMEM((NCHUNK, CHUNK), jnp.int32),
            pltpu.VMEM((NCHUNK, CHUNK), jnp.int32),
            pltpu.VMEM((NCHUNK, CHUNK), jnp.int32),
            pltpu.VMEM((NCHUNK, CHUNK, EMB), jnp.float32),
            pltpu.VMEM((NCHUNK, CHUNK, EMB), jnp.float32),
            pltpu.VMEM((NCHUNK, CHUNK, EMB), jnp.float32),
            pltpu.VMEM((NCHUNK * 128,), jnp.float32),
            pltpu.VMEM((NCHUNK, CHUNK, EMB), jnp.float32),
            pltpu.SemaphoreType.DMA,
        ],
    )
    out = f(q, qr, pid, q_table, qr_table, q_diff_table, u_flat)
    return out.reshape(B, L, EMB)


def kernel(q, qr, pid, q_table, qr_table, q_diff_table, u_table):
    q = q.astype(jnp.int32).reshape(B, NCHUNK, CHUNK)
    qr = qr.astype(jnp.int32).reshape(B, NCHUNK, CHUNK)
    pid = pid.astype(jnp.int32).reshape(B, NCHUNK, CHUNK)
    u_flat = u_table.reshape(-1)
    return _run(q, qr, pid, q_table, qr_table, q_diff_table, u_flat)
